# baseline (speedup 1.0000x reference)
import jax
import jax.numpy as jnp
from jax import lax
from jax.experimental import pallas as pl
from jax.experimental.pallas import tpu as pltpu
from jax.experimental.pallas import tpu_sc as plsc

N_ENT = 10000
N_ENT_PAD = 10240
EMB = 64
N_EDGES = 640000
CNTW = 16
NCB = N_ENT_PAD // CNTW

NC, NS = 2, 16
NW = NC * NS
E1 = 80
K = 5
C = E1 * K
ROWS_TOT = N_EDGES // C
ROWS_PT = ROWS_TOT // NW
NCHUNK = ROWS_PT
STRIPE = N_ENT_PAD // NS
CSTRIPE = NCB // NS
NREL = 31
G16 = C // 16


def _sc_body(tail_hbm, head_hbm, et_hbm, emb_hbm, wrel_hbm,
             acc_out, cnt_out,
             tidx, hidx, eidx, hs, rows, wl, hist, hmg,
             acc_sh, cnt_sh,
             sem_g0, sem_g1, sem_s0, sem_s1, sem_i0, sem_i1):
    cid = lax.axis_index("c")
    sid = lax.axis_index("s")
    wid = cid * NS + sid
    row0 = wid * ROWS_PT

    sem_g = (sem_g0, sem_g1)
    sem_s = (sem_s0, sem_s1)
    sem_i = (sem_i0, sem_i1)

    z16 = jnp.zeros((16,), jnp.float32)
    o16 = jnp.ones((16,), jnp.float32)
    iota = lax.iota(jnp.int32, 16)

    pltpu.sync_copy(wrel_hbm, wl)

    def _zh(i, c):
        hist[i, :] = z16
        return c
    lax.fori_loop(0, NCB, _zh, 0)

    for r in range(K):
        def _hm(g, c):
            hmg[r, pl.ds(g * 16, 16)] = iota + (r * 128 + g * 16)
            return c
        lax.fori_loop(0, 8, _hm, 0)

    def _zr(i, c):
        for k in range(EMB // 16):
            rows[0, i, pl.ds(k * 16, 16)] = z16
        return c
    lax.fori_loop(0, E1, _zr, 0)

    s0 = sid * STRIPE
    def _za(i, c):
        pltpu.sync_copy(rows.at[0, pl.ds(0, E1)],
                        acc_sh.at[pl.ds(s0 + i * E1, E1)])
        return c
    lax.fori_loop(0, STRIPE // E1, _za, 0)
    pltpu.sync_copy(hist.at[pl.ds(0, CSTRIPE)],
                    cnt_sh.at[pl.ds(sid * CSTRIPE, CSTRIPE)])

    plsc.subcore_barrier()

    def load_idx(c, b, sem):
        r = (row0 + c) * C
        pltpu.async_copy(tail_hbm.at[pl.ds(r, C)], tidx.at[b], sem)
        pltpu.async_copy(et_hbm.at[pl.ds(r, C)], eidx.at[b], sem)
        pltpu.async_copy(head_hbm.at[pl.ds(r, C)], hidx.at[b], sem)

    def wait_idx(b, sem):
        pltpu.make_async_copy(tail_hbm.at[pl.ds(0, C)], tidx.at[b], sem).wait()
        pltpu.make_async_copy(et_hbm.at[pl.ds(0, C)], eidx.at[b], sem).wait()
        pltpu.make_async_copy(head_hbm.at[pl.ds(0, C)], hidx.at[b], sem).wait()

    def fire_gathers(b, sem):
        pltpu.async_copy(emb_hbm.at[tidx.at[b]], rows.at[b], sem)

    def wait_gathers(b, sem):
        pltpu.make_async_copy(emb_hbm.at[tidx.at[b]], rows.at[b], sem).wait()

    def wait_scatters(b, sem):
        pltpu.make_async_copy(rows.at[b], acc_sh.at[hs.at[b]], sem).wait()

    def process(i, b):
        o = 1 - b
        c = 2 * i + b
        wait_gathers(b, sem_g[b])
        NK = EMB // 16
        for j in range(K):
            def _grp(g, cc):
                base = j * E1 + g * 16
                et16 = eidx[b, pl.ds(base, 16)]
                hv = hidx[b, pl.ds(base, 16)]
                hs[b, pl.ds(base, 16)] = hv
                plsc.addupdate_scatter(hist, [hv >> 4, hv & 15], o16)
                for l in range(0, 16, 2):
                    sa = et16[l]
                    sb = et16[l + 1]
                    ea = base + l
                    eb = ea + 1
                    ra = [rows[b, ea, pl.ds(k * 16, 16)] for k in range(NK)]
                    rb = [rows[b, eb, pl.ds(k * 16, 16)] for k in range(NK)]
                    wa = [wl[sa, pl.ds(k * 16, 16)] for k in range(NK)]
                    wb = [wl[sb, pl.ds(k * 16, 16)] for k in range(NK)]
                    for k in range(NK):
                        rows[b, ea, pl.ds(k * 16, 16)] = ra[k] * wa[k]
                    for k in range(NK):
                        rows[b, eb, pl.ds(k * 16, 16)] = rb[k] * wb[k]
                return cc
            lax.fori_loop(0, E1 // 16, _grp, 0)
            if j == 2:
                @pl.when(c >= 1)
                def _():
                    wait_scatters(o, sem_s[o])
                @pl.when(c + 1 < NCHUNK)
                def _():
                    wait_idx(o, sem_i[o])
                    fire_gathers(o, sem_g[o])
        pltpu.async_copy(rows.at[b], acc_sh.at[hs.at[b]], sem_s[b], add=True)
        @pl.when(c + 2 < NCHUNK)
        def _():
            load_idx(c + 2, b, sem_i[b])
        return None

    load_idx(0, 0, sem_i0)
    wait_idx(0, sem_i0)
    fire_gathers(0, sem_g0)
    load_idx(1, 1, sem_i1)

    def _pair(i, c):
        process(i, 0)
        process(i, 1)
        return c
    lax.fori_loop(0, NCHUNK // 2, _pair, 0)

    wait_scatters(1, sem_s1)

    for r in range(K):
        pltpu.sync_copy(hist.at[pl.ds(r * 128, 128)],
                        cnt_sh.at[hmg.at[r]], add=True)

    plsc.subcore_barrier()

    pltpu.sync_copy(acc_sh.at[pl.ds(s0, STRIPE)], acc_out.at[cid, pl.ds(s0, STRIPE)])
    pltpu.sync_copy(cnt_sh.at[pl.ds(sid * CSTRIPE, CSTRIPE)],
                    cnt_out.at[cid, pl.ds(sid * CSTRIPE, CSTRIPE)])


_sc_agg = pl.kernel(
    _sc_body,
    out_type=(
        pltpu.HBM((NC, N_ENT_PAD, EMB), jnp.float32),
        pltpu.HBM((NC, NCB, CNTW), jnp.float32),
    ),
    mesh=plsc.VectorSubcoreMesh(core_axis_name="c", subcore_axis_name="s"),
    compiler_params=pltpu.CompilerParams(use_tc_tiling_on_sc=False,
                                         needs_layout_passes=False),
    scratch_types=[
        pltpu.VMEM((2, C), jnp.int32),
        pltpu.VMEM((2, C), jnp.int32),
        pltpu.VMEM((2, C), jnp.int32),
        pltpu.VMEM((2, C), jnp.int32),
        pltpu.VMEM((2, C, EMB), jnp.float32),
        pltpu.VMEM((NREL, EMB), jnp.float32),
        pltpu.VMEM((NCB, CNTW), jnp.float32),
        pltpu.VMEM((K, 128), jnp.int32),
        pltpu.VMEM_SHARED((N_ENT_PAD, EMB), jnp.float32),
        pltpu.VMEM_SHARED((NCB, CNTW), jnp.float32),
        pltpu.SemaphoreType.DMA,
        pltpu.SemaphoreType.DMA,
        pltpu.SemaphoreType.DMA,
        pltpu.SemaphoreType.DMA,
        pltpu.SemaphoreType.DMA,
        pltpu.SemaphoreType.DMA,
    ],
)


def _combine_body(acc_ref, cnt_ref, out_ref):
    a = acc_ref[0] + acc_ref[1]
    c = cnt_ref[0] + cnt_ref[1]
    out_ref[...] = a / jnp.maximum(c, 1.0)


def _leaky(x):
    return jnp.where(x >= 0, x, 0.2 * x)


def _user_body(im_ref, emb_ref, u_ref, lat_ref, w_ref,
               w1w_ref, w1b_ref, w2w_ref, w2b_ref,
               uaw_ref, uab_ref, waw_ref, wab_ref,
               acc_ref, cnt_ref,
               uout_ref, lat_out_ref, ent_ref):
    ea = acc_ref[0] + acc_ref[1]
    ec = cnt_ref[0] + cnt_ref[1]
    ent_ref[...] = ea / jnp.maximum(ec, 1.0)
    f32 = jnp.float32
    def dott(a, b):
        return lax.dot_general(a, b, (((1,), (1,)), ((), ())),
                               preferred_element_type=f32)

    ua = jnp.dot(im_ref[...], emb_ref[...], preferred_element_type=f32)

    w1w, w1b = w1w_ref[...], w1b_ref[...]
    w2w, w2b = w2w_ref[...], w2b_ref[...]
    lat = lat_ref[...]
    w = w_ref[...]

    u1 = dott(u_ref[...], w1w) + w1b
    l1 = dott(lat, w1w) + w1b
    s2 = _leaky(dott(dott(u1, l1), uaw_ref[...]) + uab_ref[...])
    m = jnp.max(s2, axis=1, keepdims=True)
    e = jnp.exp(s2 - m)
    score = e / jnp.sum(e, axis=1, keepdims=True)

    l2 = dott(lat, w2w) + w2b
    wt2 = dott(w, w2w) + w2b
    s3 = _leaky(dott(dott(l2, wt2), waw_ref[...]) + wab_ref[...])
    m3 = jnp.max(s3, axis=1, keepdims=True)
    e3 = jnp.exp(s3 - m3)
    sm3 = e3 / jnp.sum(e3, axis=1, keepdims=True)
    latent_new = jnp.dot(sm3, w, preferred_element_type=f32)

    gate = 1.0 + jnp.dot(score, latent_new, preferred_element_type=f32)
    uout_ref[...] = ua * gate
    lat_out_ref[...] = latent_new


def kernel(entity_emb, user_emb, latent_emb, edge_index, edge_type, interact_mat,
           weight, entity_cate_set, w1_w, w1_b, w2_w, w2_b, ua_w, ua_b, wa_w, wa_b):
    n_users = user_emb.shape[0]
    n_rel1 = weight.shape[0]
    n_fac = latent_emb.shape[0]

    head = edge_index[0]
    tail = edge_index[1]
    et0 = edge_type - 1

    acc, cnt = _sc_agg(tail, head, et0, entity_emb, weight)
    cnt3 = cnt.reshape(NC, N_ENT_PAD, 1)

    BU = 256
    grid = (n_users // BU,)
    BE = N_ENT_PAD * BU // n_users
    full = lambda s: pl.BlockSpec(s, lambda i: (0, 0))
    user_agg, latent_new, entity_agg_pad = pl.pallas_call(
        _user_body,
        grid=grid,
        in_specs=[
            pl.BlockSpec((BU, N_ENT), lambda i: (i, 0)),
            full((N_ENT, EMB)),
            pl.BlockSpec((BU, EMB), lambda i: (i, 0)),
            full((n_fac, EMB)),
            full((n_rel1, EMB)),
            full(w1_w.shape),
            full((1, EMB)),
            full(w2_w.shape),
            full((1, EMB)),
            full(ua_w.shape),
            full((1, n_fac)),
            full(wa_w.shape),
            full((1, n_rel1)),
            pl.BlockSpec((NC, BE, EMB), lambda i: (0, i, 0)),
            pl.BlockSpec((NC, BE, 1), lambda i: (0, i, 0)),
        ],
        out_specs=[
            pl.BlockSpec((BU, EMB), lambda i: (i, 0)),
            pl.BlockSpec((n_fac, EMB), lambda i: (0, 0)),
            pl.BlockSpec((BE, EMB), lambda i: (i, 0)),
        ],
        out_shape=[
            jax.ShapeDtypeStruct((n_users, EMB), jnp.float32),
            jax.ShapeDtypeStruct((n_fac, EMB), jnp.float32),
            jax.ShapeDtypeStruct((N_ENT_PAD, EMB), jnp.float32),
        ],
    )(interact_mat, entity_emb, user_emb, latent_emb, weight,
      w1_w, w1_b.reshape(1, EMB), w2_w, w2_b.reshape(1, EMB),
      ua_w, ua_b.reshape(1, n_fac), wa_w, wa_b.reshape(1, n_rel1),
      acc, cnt3)

    entity_agg = entity_agg_pad[:N_ENT]

    return entity_agg, user_agg, latent_new


# speedup vs baseline: 1.0570x; 1.0570x over previous
import jax
import jax.numpy as jnp
from jax import lax
from jax.experimental import pallas as pl
from jax.experimental.pallas import tpu as pltpu
from jax.experimental.pallas import tpu_sc as plsc

N_ENT = 10000
N_ENT_PAD = 10240
EMB = 64
N_EDGES = 640000
CNTW = 16
NCB = N_ENT_PAD // CNTW

NC, NS = 2, 16
NW = NC * NS
E1 = 80
K = 5
C = E1 * K
ROWS_TOT = N_EDGES // C
ROWS_PT = ROWS_TOT // NW
NCHUNK = ROWS_PT
STRIPE = N_ENT_PAD // NS
CSTRIPE = NCB // NS
NREL = 31
G16 = C // 16


def _sc_body(tail_hbm, head_hbm, et_hbm, emb_hbm, wrel_hbm,
             acc_out, cnt_out,
             tidx, hidx, eidx, hs, rows, wl, hist, hmg,
             acc_sh, cnt_sh,
             sem_g0, sem_g1, sem_s0, sem_s1, sem_i0, sem_i1):
    cid = lax.axis_index("c")
    sid = lax.axis_index("s")
    wid = cid * NS + sid
    row0 = wid * ROWS_PT

    sem_g = (sem_g0, sem_g1)
    sem_s = (sem_s0, sem_s1)
    sem_i = (sem_i0, sem_i1)

    z16 = jnp.zeros((16,), jnp.float32)
    o16 = jnp.ones((16,), jnp.float32)
    iota = lax.iota(jnp.int32, 16)

    pltpu.sync_copy(wrel_hbm, wl)

    def _zh(i, c):
        hist[i, :] = z16
        return c
    lax.fori_loop(0, NCB, _zh, 0)

    for r in range(K):
        def _hm(g, c):
            hmg[r, pl.ds(g * 16, 16)] = iota + (r * 128 + g * 16)
            return c
        lax.fori_loop(0, 8, _hm, 0)

    def _zr(i, c):
        for k in range(EMB // 16):
            rows[0, i, pl.ds(k * 16, 16)] = z16
        return c
    lax.fori_loop(0, E1, _zr, 0)

    s0 = sid * STRIPE
    def _za(i, c):
        pltpu.sync_copy(rows.at[0, pl.ds(0, E1)],
                        acc_sh.at[pl.ds(s0 + i * E1, E1)])
        return c
    lax.fori_loop(0, STRIPE // E1, _za, 0)
    pltpu.sync_copy(hist.at[pl.ds(0, CSTRIPE)],
                    cnt_sh.at[pl.ds(sid * CSTRIPE, CSTRIPE)])

    plsc.subcore_barrier()

    def load_idx(c, b, sem):
        r = (row0 + c) * C
        pltpu.async_copy(tail_hbm.at[pl.ds(r, C)], tidx.at[b], sem)
        pltpu.async_copy(et_hbm.at[pl.ds(r, C)], eidx.at[b], sem)
        pltpu.async_copy(head_hbm.at[pl.ds(r, C)], hidx.at[b], sem)

    def wait_idx(b, sem):
        pltpu.make_async_copy(tail_hbm.at[pl.ds(0, C)], tidx.at[b], sem).wait()
        pltpu.make_async_copy(et_hbm.at[pl.ds(0, C)], eidx.at[b], sem).wait()
        pltpu.make_async_copy(head_hbm.at[pl.ds(0, C)], hidx.at[b], sem).wait()

    def fire_gathers(b, sem):
        pltpu.async_copy(emb_hbm.at[tidx.at[b]], rows.at[b], sem)

    def wait_gathers(b, sem):
        pltpu.make_async_copy(emb_hbm.at[tidx.at[b]], rows.at[b], sem).wait()

    def wait_scatters(b, sem):
        pltpu.make_async_copy(rows.at[b], acc_sh.at[hs.at[b]], sem).wait()

    def process(i, b):
        o = 1 - b
        c = 2 * i + b
        wait_gathers(b, sem_g[b])
        NK = EMB // 16
        for j in range(K):
            def _grp(g, cc):
                base = j * E1 + g * 16
                et16 = eidx[b, pl.ds(base, 16)]
                hv = hidx[b, pl.ds(base, 16)]
                hs[b, pl.ds(base, 16)] = hv
                plsc.addupdate_scatter(hist, [hv >> 4, hv & 15], o16)
                for l in range(0, 16, 2):
                    sa = et16[l]
                    sb = et16[l + 1]
                    ea = base + l
                    eb = ea + 1
                    ra = [rows[b, ea, pl.ds(k * 16, 16)] for k in range(NK)]
                    rb = [rows[b, eb, pl.ds(k * 16, 16)] for k in range(NK)]
                    wa = [wl[sa, pl.ds(k * 16, 16)] for k in range(NK)]
                    wb = [wl[sb, pl.ds(k * 16, 16)] for k in range(NK)]
                    for k in range(NK):
                        rows[b, ea, pl.ds(k * 16, 16)] = ra[k] * wa[k]
                    for k in range(NK):
                        rows[b, eb, pl.ds(k * 16, 16)] = rb[k] * wb[k]
                return cc
            lax.fori_loop(0, E1 // 16, _grp, 0)
            if j == 2:
                @pl.when(c >= 1)
                def _():
                    wait_scatters(o, sem_s[o])
                @pl.when(c + 1 < NCHUNK)
                def _():
                    wait_idx(o, sem_i[o])
                    fire_gathers(o, sem_g[o])
        pltpu.async_copy(rows.at[b], acc_sh.at[hs.at[b]], sem_s[b], add=True)
        @pl.when(c + 2 < NCHUNK)
        def _():
            load_idx(c + 2, b, sem_i[b])
        return None

    load_idx(0, 0, sem_i0)
    wait_idx(0, sem_i0)
    fire_gathers(0, sem_g0)
    load_idx(1, 1, sem_i1)

    def _pair(i, c):
        process(i, 0)
        process(i, 1)
        return c
    lax.fori_loop(0, NCHUNK // 2, _pair, 0)

    wait_scatters(1, sem_s1)

    for r in range(K):
        pltpu.sync_copy(hist.at[pl.ds(r * 128, 128)],
                        cnt_sh.at[hmg.at[r]], add=True)

    plsc.subcore_barrier()

    pltpu.sync_copy(acc_sh.at[pl.ds(s0, STRIPE)], acc_out.at[cid, pl.ds(s0, STRIPE)])
    pltpu.sync_copy(cnt_sh.at[pl.ds(sid * CSTRIPE, CSTRIPE)],
                    cnt_out.at[cid, pl.ds(sid * CSTRIPE, CSTRIPE)])


_sc_agg = pl.kernel(
    _sc_body,
    out_type=(
        pltpu.HBM((NC, N_ENT_PAD, EMB), jnp.float32),
        pltpu.HBM((NC, NCB, CNTW), jnp.float32),
    ),
    mesh=plsc.VectorSubcoreMesh(core_axis_name="c", subcore_axis_name="s"),
    compiler_params=pltpu.CompilerParams(use_tc_tiling_on_sc=False,
                                         needs_layout_passes=False),
    scratch_types=[
        pltpu.VMEM((2, C), jnp.int32),
        pltpu.VMEM((2, C), jnp.int32),
        pltpu.VMEM((2, C), jnp.int32),
        pltpu.VMEM((2, C), jnp.int32),
        pltpu.VMEM((2, C, EMB), jnp.float32),
        pltpu.VMEM((NREL, EMB), jnp.float32),
        pltpu.VMEM((NCB, CNTW), jnp.float32),
        pltpu.VMEM((K, 128), jnp.int32),
        pltpu.VMEM_SHARED((N_ENT_PAD, EMB), jnp.float32),
        pltpu.VMEM_SHARED((NCB, CNTW), jnp.float32),
        pltpu.SemaphoreType.DMA,
        pltpu.SemaphoreType.DMA,
        pltpu.SemaphoreType.DMA,
        pltpu.SemaphoreType.DMA,
        pltpu.SemaphoreType.DMA,
        pltpu.SemaphoreType.DMA,
    ],
)


def _combine_body(acc_ref, cnt_ref, out_ref):
    a = acc_ref[0] + acc_ref[1]
    c = cnt_ref[0] + cnt_ref[1]
    out_ref[...] = a / jnp.maximum(c, 1.0)


def _leaky(x):
    return jnp.where(x >= 0, x, 0.2 * x)


def _user_body(im_ref, emb_ref, u_ref, lat_ref, w_ref,
               w1w_ref, w1b_ref, w2w_ref, w2b_ref,
               uaw_ref, uab_ref, waw_ref, wab_ref,
               uout_ref, lat_out_ref):
    f32 = jnp.float32
    def dott(a, b):
        return lax.dot_general(a, b, (((1,), (1,)), ((), ())),
                               preferred_element_type=f32)

    ua = jnp.dot(im_ref[...], emb_ref[...], preferred_element_type=f32)

    w1w, w1b = w1w_ref[...], w1b_ref[...]
    w2w, w2b = w2w_ref[...], w2b_ref[...]
    lat = lat_ref[...]
    w = w_ref[...]

    u1 = dott(u_ref[...], w1w) + w1b
    l1 = dott(lat, w1w) + w1b
    s2 = _leaky(dott(dott(u1, l1), uaw_ref[...]) + uab_ref[...])
    m = jnp.max(s2, axis=1, keepdims=True)
    e = jnp.exp(s2 - m)
    score = e / jnp.sum(e, axis=1, keepdims=True)

    l2 = dott(lat, w2w) + w2b
    wt2 = dott(w, w2w) + w2b
    s3 = _leaky(dott(dott(l2, wt2), waw_ref[...]) + wab_ref[...])
    m3 = jnp.max(s3, axis=1, keepdims=True)
    e3 = jnp.exp(s3 - m3)
    sm3 = e3 / jnp.sum(e3, axis=1, keepdims=True)
    latent_new = jnp.dot(sm3, w, preferred_element_type=f32)

    gate = 1.0 + jnp.dot(score, latent_new, preferred_element_type=f32)
    uout_ref[...] = ua * gate
    lat_out_ref[...] = latent_new


def kernel(entity_emb, user_emb, latent_emb, edge_index, edge_type, interact_mat,
           weight, entity_cate_set, w1_w, w1_b, w2_w, w2_b, ua_w, ua_b, wa_w, wa_b):
    n_users = user_emb.shape[0]
    n_rel1 = weight.shape[0]
    n_fac = latent_emb.shape[0]

    head = edge_index[0]
    tail = edge_index[1]
    et0 = edge_type - 1

    BU = 256
    grid = (n_users // BU,)
    full = lambda s: pl.BlockSpec(s, lambda i: (0, 0))
    user_agg, latent_new = pl.pallas_call(
        _user_body,
        grid=grid,
        in_specs=[
            pl.BlockSpec((BU, N_ENT), lambda i: (i, 0)),
            full((N_ENT, EMB)),
            pl.BlockSpec((BU, EMB), lambda i: (i, 0)),
            full((n_fac, EMB)),
            full((n_rel1, EMB)),
            full(w1_w.shape),
            full((1, EMB)),
            full(w2_w.shape),
            full((1, EMB)),
            full(ua_w.shape),
            full((1, n_fac)),
            full(wa_w.shape),
            full((1, n_rel1)),
        ],
        out_specs=[
            pl.BlockSpec((BU, EMB), lambda i: (i, 0)),
            pl.BlockSpec((n_fac, EMB), lambda i: (0, 0)),
        ],
        out_shape=[
            jax.ShapeDtypeStruct((n_users, EMB), jnp.float32),
            jax.ShapeDtypeStruct((n_fac, EMB), jnp.float32),
        ],
    )(interact_mat, entity_emb, user_emb, latent_emb, weight,
      w1_w, w1_b.reshape(1, EMB), w2_w, w2_b.reshape(1, EMB),
      ua_w, ua_b.reshape(1, n_fac), wa_w, wa_b.reshape(1, n_rel1))

    acc, cnt = _sc_agg(tail, head, et0, entity_emb, weight)
    cnt3 = cnt.reshape(NC, N_ENT_PAD, 1)

    entity_agg_pad = pl.pallas_call(
        _combine_body,
        out_shape=jax.ShapeDtypeStruct((N_ENT_PAD, EMB), jnp.float32),
    )(acc, cnt3)
    entity_agg = entity_agg_pad[:N_ENT]

    return entity_agg, user_agg, latent_new


# speedup vs baseline: 1.0572x; 1.0002x over previous
import jax
import jax.numpy as jnp
from jax import lax
from jax.experimental import pallas as pl
from jax.experimental.pallas import tpu as pltpu
from jax.experimental.pallas import tpu_sc as plsc

N_ENT = 10000
N_ENT_PAD = 10240
EMB = 64
N_EDGES = 640000
CNTW = 16
NCB = N_ENT_PAD // CNTW

NC, NS = 2, 16
NW = NC * NS
E1 = 80
K = 5
C = E1 * K
ROWS_TOT = N_EDGES // C
ROWS_PT = ROWS_TOT // NW
NCHUNK = ROWS_PT
STRIPE = N_ENT_PAD // NS
CSTRIPE = NCB // NS
NREL = 31


def _sc_body(tail_hbm, head_hbm, et_hbm, emb_hbm, wrel_hbm,
             acc_out, cnt_out,
             tidx, hidx, eidx, hs, rows, wl, hist, hmg,
             acc_sh, cnt_sh,
             sem_g0, sem_g1, sem_s0, sem_s1, sem_i0, sem_i1):
    cid = lax.axis_index("c")
    sid = lax.axis_index("s")
    wid = cid * NS + sid
    row0 = wid * ROWS_PT

    sem_g = (sem_g0, sem_g1)
    sem_s = (sem_s0, sem_s1)
    sem_i = (sem_i0, sem_i1)

    z16 = jnp.zeros((16,), jnp.float32)
    o16 = jnp.ones((16,), jnp.float32)
    iota = lax.iota(jnp.int32, 16)

    pltpu.sync_copy(wrel_hbm, wl)

    def _zh(i, c):
        hist[i, :] = z16
        return c
    lax.fori_loop(0, NCB, _zh, 0)

    for r in range(K):
        def _hm(g, c):
            hmg[r, pl.ds(g * 16, 16)] = iota + (r * 128 + g * 16)
            return c
        lax.fori_loop(0, 8, _hm, 0)

    def _zr(i, c):
        for k in range(EMB // 16):
            rows[0, i, pl.ds(k * 16, 16)] = z16
        return c
    lax.fori_loop(0, E1, _zr, 0)

    s0 = sid * STRIPE
    def _za(i, c):
        pltpu.sync_copy(rows.at[0, pl.ds(0, E1)],
                        acc_sh.at[pl.ds(s0 + i * E1, E1)])
        return c
    lax.fori_loop(0, STRIPE // E1, _za, 0)
    pltpu.sync_copy(hist.at[pl.ds(0, CSTRIPE)],
                    cnt_sh.at[pl.ds(sid * CSTRIPE, CSTRIPE)])

    plsc.subcore_barrier()

    def load_idx(c, b, sem):
        r = (row0 + c) * C
        pltpu.async_copy(tail_hbm.at[pl.ds(r, C)], tidx.at[b], sem)
        pltpu.async_copy(et_hbm.at[pl.ds(r, C)], eidx.at[b], sem)
        pltpu.async_copy(head_hbm.at[pl.ds(r, C)], hidx.at[b], sem)

    def wait_idx(b, sem):
        pltpu.make_async_copy(tail_hbm.at[pl.ds(0, C)], tidx.at[b], sem).wait()
        pltpu.make_async_copy(et_hbm.at[pl.ds(0, C)], eidx.at[b], sem).wait()
        pltpu.make_async_copy(head_hbm.at[pl.ds(0, C)], hidx.at[b], sem).wait()

    def fire_gathers(b, sem):
        pltpu.async_copy(emb_hbm.at[tidx.at[b]], rows.at[b], sem)

    def wait_gathers(b, sem):
        pltpu.make_async_copy(emb_hbm.at[tidx.at[b]], rows.at[b], sem).wait()

    def wait_scatters(b, sem):
        pltpu.make_async_copy(rows.at[b], acc_sh.at[hs.at[b]], sem).wait()

    def process(i, b):
        o = 1 - b
        c = 2 * i + b
        wait_gathers(b, sem_g[b])
        NK = EMB // 16
        for j in range(K):
            def _grp(g, cc):
                base = j * E1 + g * 16
                et16 = eidx[b, pl.ds(base, 16)]
                hv = hidx[b, pl.ds(base, 16)]
                hs[b, pl.ds(base, 16)] = hv
                plsc.addupdate_scatter(hist, [hv >> 4, hv & 15], o16)
                for l in range(0, 16, 2):
                    sa = et16[l]
                    sb = et16[l + 1]
                    ea = base + l
                    eb = ea + 1
                    ra = [rows[b, ea, pl.ds(k * 16, 16)] for k in range(NK)]
                    rb = [rows[b, eb, pl.ds(k * 16, 16)] for k in range(NK)]
                    wa = [wl[sa, pl.ds(k * 16, 16)] for k in range(NK)]
                    wb = [wl[sb, pl.ds(k * 16, 16)] for k in range(NK)]
                    for k in range(NK):
                        rows[b, ea, pl.ds(k * 16, 16)] = ra[k] * wa[k]
                    for k in range(NK):
                        rows[b, eb, pl.ds(k * 16, 16)] = rb[k] * wb[k]
                return cc
            lax.fori_loop(0, E1 // 16, _grp, 0)
            if j == 2:
                @pl.when(c >= 1)
                def _():
                    wait_scatters(o, sem_s[o])
                @pl.when(c + 1 < NCHUNK)
                def _():
                    wait_idx(o, sem_i[o])
                    fire_gathers(o, sem_g[o])
        pltpu.async_copy(rows.at[b], acc_sh.at[hs.at[b]], sem_s[b], add=True)
        @pl.when(c + 2 < NCHUNK)
        def _():
            load_idx(c + 2, b, sem_i[b])
        return None

    load_idx(0, 0, sem_i0)
    wait_idx(0, sem_i0)
    fire_gathers(0, sem_g0)
    load_idx(1, 1, sem_i1)

    def _pair(i, c):
        process(i, 0)
        process(i, 1)
        return c
    lax.fori_loop(0, NCHUNK // 2, _pair, 0)

    wait_scatters(1, sem_s1)

    for r in range(K):
        pltpu.sync_copy(hist.at[pl.ds(r * 128, 128)],
                        cnt_sh.at[hmg.at[r]], add=True)

    plsc.subcore_barrier()

    pltpu.sync_copy(acc_sh.at[pl.ds(s0, STRIPE)], acc_out.at[cid, pl.ds(s0, STRIPE)])
    pltpu.sync_copy(cnt_sh.at[pl.ds(sid * CSTRIPE, CSTRIPE)],
                    cnt_out.at[cid, pl.ds(sid * CSTRIPE, CSTRIPE)])


_sc_agg = pl.kernel(
    _sc_body,
    out_type=(
        pltpu.HBM((NC, N_ENT_PAD, EMB), jnp.float32),
        pltpu.HBM((NC, NCB, CNTW), jnp.float32),
    ),
    mesh=plsc.VectorSubcoreMesh(core_axis_name="c", subcore_axis_name="s"),
    compiler_params=pltpu.CompilerParams(use_tc_tiling_on_sc=False,
                                         needs_layout_passes=False),
    scratch_types=[
        pltpu.VMEM((2, C), jnp.int32),
        pltpu.VMEM((2, C), jnp.int32),
        pltpu.VMEM((2, C), jnp.int32),
        pltpu.VMEM((2, C), jnp.int32),
        pltpu.VMEM((2, C, EMB), jnp.float32),
        pltpu.VMEM((NREL, EMB), jnp.float32),
        pltpu.VMEM((NCB, CNTW), jnp.float32),
        pltpu.VMEM((K, 128), jnp.int32),
        pltpu.VMEM_SHARED((N_ENT_PAD, EMB), jnp.float32),
        pltpu.VMEM_SHARED((NCB, CNTW), jnp.float32),
        pltpu.SemaphoreType.DMA,
        pltpu.SemaphoreType.DMA,
        pltpu.SemaphoreType.DMA,
        pltpu.SemaphoreType.DMA,
        pltpu.SemaphoreType.DMA,
        pltpu.SemaphoreType.DMA,
    ],
)


def _combine_body(acc_ref, cnt_ref, out_ref):
    a = acc_ref[0] + acc_ref[1]
    c = cnt_ref[0] + cnt_ref[1]
    out_ref[...] = a / jnp.maximum(c, 1.0)


def _leaky(x):
    return jnp.where(x >= 0, x, 0.2 * x)


def _user_body(im_ref, emb_ref, u_ref, lat_ref, w_ref,
               w1w_ref, w1b_ref, w2w_ref, w2b_ref,
               uaw_ref, uab_ref, waw_ref, wab_ref,
               uout_ref, lat_out_ref):
    f32 = jnp.float32
    def dott(a, b):
        return lax.dot_general(a, b, (((1,), (1,)), ((), ())),
                               preferred_element_type=f32)

    ua = jnp.dot(im_ref[...], emb_ref[...], preferred_element_type=f32)

    w1w, w1b = w1w_ref[...], w1b_ref[...]
    w2w, w2b = w2w_ref[...], w2b_ref[...]
    lat = lat_ref[...]
    w = w_ref[...]

    u1 = dott(u_ref[...], w1w) + w1b
    l1 = dott(lat, w1w) + w1b
    s2 = _leaky(dott(dott(u1, l1), uaw_ref[...]) + uab_ref[...])
    m = jnp.max(s2, axis=1, keepdims=True)
    e = jnp.exp(s2 - m)
    score = e / jnp.sum(e, axis=1, keepdims=True)

    l2 = dott(lat, w2w) + w2b
    wt2 = dott(w, w2w) + w2b
    s3 = _leaky(dott(dott(l2, wt2), waw_ref[...]) + wab_ref[...])
    m3 = jnp.max(s3, axis=1, keepdims=True)
    e3 = jnp.exp(s3 - m3)
    sm3 = e3 / jnp.sum(e3, axis=1, keepdims=True)
    latent_new = jnp.dot(sm3, w, preferred_element_type=f32)

    gate = 1.0 + jnp.dot(score, latent_new, preferred_element_type=f32)
    uout_ref[...] = ua * gate
    lat_out_ref[...] = latent_new


def kernel(entity_emb, user_emb, latent_emb, edge_index, edge_type, interact_mat,
           weight, entity_cate_set, w1_w, w1_b, w2_w, w2_b, ua_w, ua_b, wa_w, wa_b):
    n_users = user_emb.shape[0]
    n_rel1 = weight.shape[0]
    n_fac = latent_emb.shape[0]

    head = edge_index[0]
    tail = edge_index[1]
    et0 = edge_type - 1

    BU = 256
    grid = (n_users // BU,)
    full = lambda s: pl.BlockSpec(s, lambda i: (0, 0))
    user_agg, latent_new = pl.pallas_call(
        _user_body,
        grid=grid,
        in_specs=[
            pl.BlockSpec((BU, N_ENT), lambda i: (i, 0)),
            full((N_ENT, EMB)),
            pl.BlockSpec((BU, EMB), lambda i: (i, 0)),
            full((n_fac, EMB)),
            full((n_rel1, EMB)),
            full(w1_w.shape),
            full((1, EMB)),
            full(w2_w.shape),
            full((1, EMB)),
            full(ua_w.shape),
            full((1, n_fac)),
            full(wa_w.shape),
            full((1, n_rel1)),
        ],
        out_specs=[
            pl.BlockSpec((BU, EMB), lambda i: (i, 0)),
            pl.BlockSpec((n_fac, EMB), lambda i: (0, 0)),
        ],
        out_shape=[
            jax.ShapeDtypeStruct((n_users, EMB), jnp.float32),
            jax.ShapeDtypeStruct((n_fac, EMB), jnp.float32),
        ],
    )(interact_mat, entity_emb, user_emb, latent_emb, weight,
      w1_w, w1_b.reshape(1, EMB), w2_w, w2_b.reshape(1, EMB),
      ua_w, ua_b.reshape(1, n_fac), wa_w, wa_b.reshape(1, n_rel1))

    acc, cnt = _sc_agg(tail, head, et0, entity_emb, weight)
    cnt3 = cnt.reshape(NC, N_ENT_PAD, 1)

    entity_agg_pad = pl.pallas_call(
        _combine_body,
        out_shape=jax.ShapeDtypeStruct((N_ENT_PAD, EMB), jnp.float32),
    )(acc, cnt3)
    entity_agg = entity_agg_pad[:N_ENT]

    return entity_agg, user_agg, latent_new
